# baseline (device time: 20972 ns/iter reference)
import jax
import jax.numpy as jnp
from jax import lax
from jax.experimental import pallas as pl
from jax.experimental.pallas import tpu as pltpu

N_DEV = 8
N_WQ = 4
_GELU_C = 0.7978845608028654


_POS2C = [(0, 0, 0), (1, 0, 0), (1, 1, 0), (0, 1, 0),
          (0, 0, 1), (1, 0, 1), (1, 1, 1), (0, 1, 1)]
_C2POS = {c: i for i, c in enumerate(_POS2C)}
_CLASSES = [(1, 0, 0), (1, 1, 0), (1, 0, 1), (1, 1, 1),
            (0, 1, 0), (0, 1, 1), (0, 0, 1)]
_DEST = [
    [_C2POS[tuple(a ^ b for a, b in zip(_POS2C[m], cls))] for cls in _CLASSES]
    for m in range(8)
]


def _lut(idx, col):
    v = jnp.int32(col[0])
    for m in range(1, 8):
        v = jnp.where(idx == m, jnp.int32(col[m]), v)
    return v

def _gelu_bf16(y):
    y = 0.5 * y * (1.0 + jnp.tanh(_GELU_C * (y + 0.044715 * y * y * y)))
    return y.astype(jnp.bfloat16)


def kernel(x, w_mat):
    m_per, k = x.shape
    _, n = w_mat.shape
    blk = n // N_DEV
    kq = k // N_WQ

    def body(x_hbm, w_hbm, out_ref, xv_ref, w_ref, y_ref, part_ref, own_sem,
             x_sem, w_sems, send_sems, recv_sems):
        my_i = lax.axis_index("i")

        x_cp = pltpu.make_async_copy(x_hbm, xv_ref, x_sem)
        x_cp.start()
        w_cps = []
        for q in range(N_WQ):
            cp = pltpu.make_async_copy(
                w_hbm.at[pl.ds(q * kq, kq), :],
                w_ref.at[pl.ds(q * kq, kq), :],
                w_sems.at[q],
            )
            cp.start()
            w_cps.append(cp)

        barrier_sem = pltpu.get_barrier_semaphore()
        for j in range(1, N_DEV):
            pl.semaphore_signal(
                barrier_sem, inc=1,
                device_id=((my_i + j) % N_DEV,),
                device_id_type=pl.DeviceIdType.MESH,
            )
        pl.semaphore_wait(barrier_sem, N_DEV - 1)

        x_cp.wait()
        kh = k // 2
        w_cps[0].wait()
        w_cps[1].wait()

        for t in range(N_DEV - 1):
            p = _lut(my_i, [_DEST[m][t] for m in range(N_DEV)])
            part_ref[t + 1, :, :] = jnp.dot(
                xv_ref[:, :kh], w_ref[:kh, pl.ds(p * blk, blk)],
                preferred_element_type=jnp.float32)
        part_ref[0, :, :] = jnp.dot(
            xv_ref[:, :kh], w_ref[:kh, pl.ds(my_i * blk, blk)],
            preferred_element_type=jnp.float32)

        w_cps[2].wait()
        w_cps[3].wait()

        sends = []
        for t in range(N_DEV - 1):
            p = _lut(my_i, [_DEST[m][t] for m in range(N_DEV)])
            y_ref[t + 1, :, :] = _gelu_bf16(
                part_ref[t + 1, :, :]
                + jnp.dot(xv_ref[:, kh:], w_ref[kh:, pl.ds(p * blk, blk)],
                          preferred_element_type=jnp.float32)
            )
            rdma = pltpu.make_async_remote_copy(
                src_ref=y_ref.at[t + 1],
                dst_ref=out_ref.at[pl.ds(my_i * m_per, m_per), :],
                send_sem=send_sems.at[t + 1],
                recv_sem=recv_sems.at[t + 1],
                device_id=(p,),
                device_id_type=pl.DeviceIdType.MESH,
            )
            rdma.start()
            sends.append(rdma)

        y_ref[0, :, :] = _gelu_bf16(
            part_ref[0, :, :]
            + jnp.dot(xv_ref[:, kh:], w_ref[kh:, pl.ds(my_i * blk, blk)],
                      preferred_element_type=jnp.float32)
        )
        own_cp = pltpu.make_async_copy(
            y_ref.at[0],
            out_ref.at[pl.ds(my_i * m_per, m_per), :],
            own_sem,
        )
        own_cp.start()

        for t in range(N_DEV - 1):
            s = _lut(my_i, [_DEST[m][t] for m in range(N_DEV)])
            recv = pltpu.make_async_remote_copy(
                src_ref=y_ref.at[t + 1],
                dst_ref=out_ref.at[pl.ds(s * m_per, m_per), :],
                send_sem=send_sems.at[t + 1],
                recv_sem=recv_sems.at[t + 1],
                device_id=(s,),
                device_id_type=pl.DeviceIdType.MESH,
            )
            recv.wait_recv()

        own_cp.wait()
        for rdma in sends:
            rdma.wait_send()

    x = pltpu.with_memory_space_constraint(x, pltpu.MemorySpace.HBM)
    w_mat = pltpu.with_memory_space_constraint(w_mat, pltpu.MemorySpace.HBM)
    out_shape = jax.ShapeDtypeStruct((N_DEV * m_per, blk), jnp.bfloat16)
    return pl.pallas_call(
        body,
        out_shape=out_shape,
        in_specs=[
            pl.BlockSpec(memory_space=pltpu.MemorySpace.HBM),
            pl.BlockSpec(memory_space=pltpu.MemorySpace.HBM),
        ],
        out_specs=pl.BlockSpec(memory_space=pltpu.MemorySpace.HBM),
        scratch_shapes=[
            pltpu.VMEM((m_per, k), jnp.float32),
            pltpu.VMEM((k, n), jnp.float32),
            pltpu.VMEM((N_DEV, m_per, blk), jnp.bfloat16),
            pltpu.VMEM((N_DEV, m_per, blk), jnp.float32),
            pltpu.SemaphoreType.DMA,
            pltpu.SemaphoreType.DMA,
            pltpu.SemaphoreType.DMA((N_WQ,)),
            pltpu.SemaphoreType.DMA((N_DEV,)),
            pltpu.SemaphoreType.DMA((N_DEV,)),
        ],
        compiler_params=pltpu.CompilerParams(collective_id=0),
    )(x, w_mat)


# device time: 19917 ns/iter; 1.0530x vs baseline; 1.0530x over previous
import jax
import jax.numpy as jnp
from jax import lax
from jax.experimental import pallas as pl
from jax.experimental.pallas import tpu as pltpu

N_DEV = 8
N_WQ = 4
_GELU_C = 0.7978845608028654


def _gelu_bf16(y):
    y = 0.5 * y * (1.0 + jnp.tanh(_GELU_C * (y + 0.044715 * y * y * y)))
    return y.astype(jnp.bfloat16)


def kernel(x, w_mat):
    m_per, k = x.shape
    _, n = w_mat.shape
    blk = n // N_DEV
    kq = k // N_WQ

    def body(x_hbm, w_hbm, out_ref, xv_ref, w_ref, wcol_ref, y_ref, part_ref, own_sem, wc_sems,
             x_sem, w_sems, send_sems, recv_sems):
        my_i = lax.axis_index("i")

        x_cp = pltpu.make_async_copy(x_hbm, xv_ref, x_sem)
        x_cp.start()
        wc_cps = []
        for e in range(2):
            p = (my_i + 1 + e) % N_DEV
            cp = pltpu.make_async_copy(
                w_hbm.at[:, pl.ds(p * blk, blk)],
                wcol_ref.at[e],
                wc_sems.at[e],
            )
            cp.start()
            wc_cps.append(cp)
        w_cps = []
        for q in range(N_WQ):
            cp = pltpu.make_async_copy(
                w_hbm.at[pl.ds(q * kq, kq), :],
                w_ref.at[pl.ds(q * kq, kq), :],
                w_sems.at[q],
            )
            cp.start()
            w_cps.append(cp)

        barrier_sem = pltpu.get_barrier_semaphore()
        for j in range(1, N_DEV):
            pl.semaphore_signal(
                barrier_sem, inc=1,
                device_id=((my_i + j) % N_DEV,),
                device_id_type=pl.DeviceIdType.MESH,
            )
        pl.semaphore_wait(barrier_sem, N_DEV - 1)

        x_cp.wait()
        kh = k // 2

        sends = []
        for e in range(2):
            j = e + 1
            p = (my_i + j) % N_DEV
            wc_cps[e].wait()
            y_ref[j, :, :] = _gelu_bf16(
                jnp.dot(xv_ref[...], wcol_ref[e],
                        preferred_element_type=jnp.float32)
            )
            rdma = pltpu.make_async_remote_copy(
                src_ref=y_ref.at[j],
                dst_ref=out_ref.at[pl.ds(my_i * m_per, m_per), :],
                send_sem=send_sems.at[j],
                recv_sem=recv_sems.at[j],
                device_id=(p,),
                device_id_type=pl.DeviceIdType.MESH,
            )
            rdma.start()
            sends.append(rdma)

        w_cps[0].wait()
        w_cps[1].wait()

        for j in range(3, N_DEV):
            p = (my_i + j) % N_DEV
            part_ref[j, :, :] = jnp.dot(
                xv_ref[:, :kh], w_ref[:kh, pl.ds(p * blk, blk)],
                preferred_element_type=jnp.float32)
        part_ref[0, :, :] = jnp.dot(
            xv_ref[:, :kh], w_ref[:kh, pl.ds(my_i * blk, blk)],
            preferred_element_type=jnp.float32)

        w_cps[2].wait()
        w_cps[3].wait()

        for j in range(3, N_DEV):
            p = (my_i + j) % N_DEV
            y_ref[j, :, :] = _gelu_bf16(
                part_ref[j, :, :]
                + jnp.dot(xv_ref[:, kh:], w_ref[kh:, pl.ds(p * blk, blk)],
                          preferred_element_type=jnp.float32)
            )
            rdma = pltpu.make_async_remote_copy(
                src_ref=y_ref.at[j],
                dst_ref=out_ref.at[pl.ds(my_i * m_per, m_per), :],
                send_sem=send_sems.at[j],
                recv_sem=recv_sems.at[j],
                device_id=(p,),
                device_id_type=pl.DeviceIdType.MESH,
            )
            rdma.start()
            sends.append(rdma)

        y_ref[0, :, :] = _gelu_bf16(
            part_ref[0, :, :]
            + jnp.dot(xv_ref[:, kh:], w_ref[kh:, pl.ds(my_i * blk, blk)],
                      preferred_element_type=jnp.float32)
        )
        own_cp = pltpu.make_async_copy(
            y_ref.at[0],
            out_ref.at[pl.ds(my_i * m_per, m_per), :],
            own_sem,
        )
        own_cp.start()

        for j in range(1, N_DEV):
            s = (my_i - j) % N_DEV
            recv = pltpu.make_async_remote_copy(
                src_ref=y_ref.at[j],
                dst_ref=out_ref.at[pl.ds(s * m_per, m_per), :],
                send_sem=send_sems.at[j],
                recv_sem=recv_sems.at[j],
                device_id=(s,),
                device_id_type=pl.DeviceIdType.MESH,
            )
            recv.wait_recv()

        own_cp.wait()
        for rdma in sends:
            rdma.wait_send()

    x = pltpu.with_memory_space_constraint(x, pltpu.MemorySpace.HBM)
    w_mat = pltpu.with_memory_space_constraint(w_mat, pltpu.MemorySpace.HBM)
    out_shape = jax.ShapeDtypeStruct((N_DEV * m_per, blk), jnp.bfloat16)
    return pl.pallas_call(
        body,
        out_shape=out_shape,
        in_specs=[
            pl.BlockSpec(memory_space=pltpu.MemorySpace.HBM),
            pl.BlockSpec(memory_space=pltpu.MemorySpace.HBM),
        ],
        out_specs=pl.BlockSpec(memory_space=pltpu.MemorySpace.HBM),
        scratch_shapes=[
            pltpu.VMEM((m_per, k), jnp.float32),
            pltpu.VMEM((k, n), jnp.float32),
            pltpu.VMEM((2, k, blk), jnp.float32),
            pltpu.VMEM((N_DEV, m_per, blk), jnp.bfloat16),
            pltpu.VMEM((N_DEV, m_per, blk), jnp.float32),
            pltpu.SemaphoreType.DMA,
            pltpu.SemaphoreType.DMA((2,)),
            pltpu.SemaphoreType.DMA,
            pltpu.SemaphoreType.DMA((N_WQ,)),
            pltpu.SemaphoreType.DMA((N_DEV,)),
            pltpu.SemaphoreType.DMA((N_DEV,)),
        ],
        compiler_params=pltpu.CompilerParams(collective_id=0),
    )(x, w_mat)


# device time: 18933 ns/iter; 1.1077x vs baseline; 1.0520x over previous
import jax
import jax.numpy as jnp
from jax import lax
from jax.experimental import pallas as pl
from jax.experimental.pallas import tpu as pltpu

N_DEV = 8
N_WQ = 4
_GELU_C = 0.7978845608028654


def _gelu_bf16(y):
    y = 0.5 * y * (1.0 + jnp.tanh(_GELU_C * (y + 0.044715 * y * y * y)))
    return y.astype(jnp.bfloat16)


def kernel(x, w_mat):
    m_per, k = x.shape
    _, n = w_mat.shape
    blk = n // N_DEV
    kq = k // N_WQ

    def body(x_hbm, w_hbm, out_ref, xv_ref, w_ref, y_ref, part_ref, own_sem,
             x_sem, w_sems, send_sems, recv_sems):
        my_i = lax.axis_index("i")

        x_cp = pltpu.make_async_copy(x_hbm, xv_ref, x_sem)
        x_cp.start()
        w_cps = []
        for q in range(N_WQ):
            cp = pltpu.make_async_copy(
                w_hbm.at[pl.ds(q * kq, kq), :],
                w_ref.at[pl.ds(q * kq, kq), :],
                w_sems.at[q],
            )
            cp.start()
            w_cps.append(cp)

        barrier_sem = pltpu.get_barrier_semaphore()
        for j in range(1, N_DEV):
            pl.semaphore_signal(
                barrier_sem, inc=1,
                device_id=((my_i + j) % N_DEV,),
                device_id_type=pl.DeviceIdType.MESH,
            )
        pl.semaphore_wait(barrier_sem, N_DEV - 1)

        x_cp.wait()
        kh = k // 2
        w_cps[0].wait()
        w_cps[1].wait()

        for j in range(N_DEV):
            p = (my_i + j) % N_DEV
            part_ref[j, :, :] = jnp.dot(
                xv_ref[:, :kh], w_ref[:kh, pl.ds(p * blk, blk)],
                preferred_element_type=jnp.float32)

        w_cps[2].wait()
        w_cps[3].wait()

        sends = []
        for j in range(1, N_DEV):
            p = (my_i + j) % N_DEV
            y_ref[j, :, :] = _gelu_bf16(
                part_ref[j, :, :]
                + jnp.dot(xv_ref[:, kh:], w_ref[kh:, pl.ds(p * blk, blk)],
                          preferred_element_type=jnp.float32)
            )
            rdma = pltpu.make_async_remote_copy(
                src_ref=y_ref.at[j],
                dst_ref=out_ref.at[pl.ds(my_i * m_per, m_per), :],
                send_sem=send_sems.at[j],
                recv_sem=recv_sems.at[j],
                device_id=(p,),
                device_id_type=pl.DeviceIdType.MESH,
            )
            rdma.start()
            sends.append(rdma)

        y_ref[0, :, :] = _gelu_bf16(
            part_ref[0, :, :]
            + jnp.dot(xv_ref[:, kh:], w_ref[kh:, pl.ds(my_i * blk, blk)],
                      preferred_element_type=jnp.float32)
        )
        own_cp = pltpu.make_async_copy(
            y_ref.at[0],
            out_ref.at[pl.ds(my_i * m_per, m_per), :],
            own_sem,
        )
        own_cp.start()

        for j in range(1, N_DEV):
            s = (my_i - j) % N_DEV
            recv = pltpu.make_async_remote_copy(
                src_ref=y_ref.at[j],
                dst_ref=out_ref.at[pl.ds(s * m_per, m_per), :],
                send_sem=send_sems.at[j],
                recv_sem=recv_sems.at[j],
                device_id=(s,),
                device_id_type=pl.DeviceIdType.MESH,
            )
            recv.wait_recv()

        own_cp.wait()
        for rdma in sends:
            rdma.wait_send()

    x = pltpu.with_memory_space_constraint(x, pltpu.MemorySpace.HBM)
    w_mat = pltpu.with_memory_space_constraint(w_mat, pltpu.MemorySpace.HBM)
    out_shape = jax.ShapeDtypeStruct((N_DEV * m_per, blk), jnp.bfloat16)
    return pl.pallas_call(
        body,
        out_shape=out_shape,
        in_specs=[
            pl.BlockSpec(memory_space=pltpu.MemorySpace.HBM),
            pl.BlockSpec(memory_space=pltpu.MemorySpace.HBM),
        ],
        out_specs=pl.BlockSpec(memory_space=pltpu.MemorySpace.HBM),
        scratch_shapes=[
            pltpu.VMEM((m_per, k), jnp.float32),
            pltpu.VMEM((k, n), jnp.float32),
            pltpu.VMEM((N_DEV, m_per, blk), jnp.bfloat16),
            pltpu.VMEM((N_DEV, m_per, blk), jnp.float32),
            pltpu.SemaphoreType.DMA,
            pltpu.SemaphoreType.DMA,
            pltpu.SemaphoreType.DMA((N_WQ,)),
            pltpu.SemaphoreType.DMA((N_DEV,)),
            pltpu.SemaphoreType.DMA((N_DEV,)),
        ],
        compiler_params=pltpu.CompilerParams(collective_id=0),
    )(x, w_mat)
